# baseline (speedup 1.0000x reference)
import jax
import jax.numpy as jnp
from jax.experimental import pallas as pl
from jax.experimental.pallas import tpu as pltpu

_EPS = 1e-5
_B = 16
_D = 32
_CX = 29
_T = 2048


def _body(o_ref, x_ref, p_ref, a0x_ref, a0p_ref, b0_ref, a1_ref, b1_ref,
          out_ref):
    g = pl.program_id(0)
    nsteps = pl.num_programs(0)

    @pl.when(g == 0)
    def _init():
        out_ref[:] = jnp.full_like(out_ref, -jnp.inf)

    offs = [o_ref[j] for j in range(_B)]
    row0 = g * _T
    row_last = row0 + _T - 1
    s0 = sum(jnp.where(offs[j] <= row0, 1, 0) for j in range(_B))
    s1 = sum(jnp.where(offs[j] <= row_last, 1, 0) for j in range(_B))

    @pl.when(s0 < _B)
    def _compute():
        h = (jnp.dot(x_ref[:], a0x_ref[:], preferred_element_type=jnp.float32)
             + jnp.dot(p_ref[:], a0p_ref[:], preferred_element_type=jnp.float32)
             + b0_ref[:])
        h = jnp.maximum(h, 0.0)
        h = jnp.dot(h, a1_ref[:], preferred_element_type=jnp.float32) + b1_ref[:]
        h = jnp.maximum(h, 0.0)

        seg_iota = jax.lax.broadcasted_iota(jnp.int32, (_B, 1), 0)

        fast = s0 == s1

        @pl.when(fast)
        def _whole_tile_one_segment():
            m = jnp.max(h, axis=0)
            sel = seg_iota == s0
            out_ref[:] = jnp.where(sel, jnp.maximum(out_ref[:], m[None, :]),
                                   out_ref[:])

        @pl.when(jnp.logical_not(fast))
        def _straddles_boundaries():
            rows = row0 + jax.lax.broadcasted_iota(jnp.int32, (_T, 1), 0)
            for i in range(_B):
                @pl.when(jnp.logical_and(i >= s0, i <= s1))
                def _one_segment(i=i):
                    start = offs[i - 1] if i > 0 else jnp.int32(0)
                    end = offs[i]
                    mask = jnp.logical_and(rows >= start, rows < end)
                    m = jnp.max(jnp.where(mask, h, -jnp.inf), axis=0)
                    sel = seg_iota == i
                    out_ref[:] = jnp.where(
                        sel, jnp.maximum(out_ref[:], m[None, :]), out_ref[:])

    @pl.when(g == nsteps - 1)
    def _finalize():
        out_ref[:] = jnp.maximum(out_ref[:], 0.0)


def kernel(p, x, o, W0, gamma0, beta0, W1, gamma1, beta1):
    n = x.shape[0]
    nsteps = n // _T
    s = 1.0 / jnp.sqrt(jnp.float32(1.0) + _EPS)
    a0 = W0.T * (gamma0 * s)[None, :]
    a0x = a0[:_CX]
    a0p = a0[_CX:]
    b0 = beta0.reshape(1, _D)
    a1 = W1.T * (gamma1 * s)[None, :]
    b1 = beta1.reshape(1, _D)

    def _row_map(i, o_ref):
        last_blk = jnp.maximum((o_ref[_B - 1] - 1) // _T, 0)
        return (jnp.minimum(i, last_blk), 0)

    def _fixed(i, o_ref):
        return (0, 0)

    grid_spec = pltpu.PrefetchScalarGridSpec(
        num_scalar_prefetch=1,
        grid=(nsteps,),
        in_specs=[
            pl.BlockSpec((_T, _CX), _row_map),
            pl.BlockSpec((_T, 3), _row_map),
            pl.BlockSpec((_CX, _D), _fixed),
            pl.BlockSpec((3, _D), _fixed),
            pl.BlockSpec((1, _D), _fixed),
            pl.BlockSpec((_D, _D), _fixed),
            pl.BlockSpec((1, _D), _fixed),
        ],
        out_specs=pl.BlockSpec((_B, _D), _fixed),
    )
    n_x = pl.pallas_call(
        _body,
        grid_spec=grid_spec,
        out_shape=jax.ShapeDtypeStruct((_B, _D), jnp.float32),
    )(o, x, p, a0x, a0p, b0, a1, b1)

    n_p = jnp.zeros((_B, 3), dtype=p.dtype)
    n_o = jnp.arange(_B, dtype=o.dtype) + 1
    return (n_p, n_x, n_o)
